# Initial kernel scaffold; baseline (speedup 1.0000x reference)
import jax
import jax.numpy as jnp
from jax.experimental import pallas as pl


def kernel(x, edge_index, edge_weight, W0, b0, gamma0, beta0, mean0, var0, W1, b1):
    raise NotImplementedError("write your pallas kernel here")


# speedup vs baseline: 3.9188x; 3.9188x over previous
import functools

import jax
import jax.numpy as jnp
from jax import lax
from jax.experimental import pallas as pl
from jax.experimental.pallas import tpu as pltpu
from jax.experimental.pallas import tpu_sc as plsc

N = 10000
E = 320000
D = 128
H = 128

NC = 2
NS = 16
L = 16

E_CORE = E // NC
E_SUB = E_CORE // NS
CHUNK = 80
NCHUNK = E_SUB // CHUNK
GROUPS = CHUNK // L
N_PAD = 10240
N_SUB = N_PAD // NS
ZROWS = 128


def _spmm_body(h_hbm, src_hbm, dst_hbm, ew_hbm, out_hbm,
               src_v, dst_v, ew_v, rows_v, stage_v, acc_sh, sem):
    cid = lax.axis_index("c")
    sid = lax.axis_index("s")

    zvec = jnp.zeros((L,), jnp.float32)

    @pl.loop(0, ZROWS)
    def _zero_stage(r):
        for j in range(H // L):
            stage_v[r, pl.ds(j * L, L)] = zvec

    for k in range(N_SUB // ZROWS):
        pltpu.sync_copy(stage_v,
                        acc_sh.at[pl.ds(sid * N_SUB + k * ZROWS, ZROWS)])
    plsc.subcore_barrier()

    base = cid * E_CORE + sid * E_SUB

    @pl.loop(0, NCHUNK)
    def _chunk(k):
        off = base + k * CHUNK
        pltpu.sync_copy(src_hbm.at[pl.ds(off, CHUNK)], src_v)
        pltpu.sync_copy(dst_hbm.at[pl.ds(off, CHUNK)], dst_v)
        pltpu.sync_copy(ew_hbm.at[pl.ds(off, CHUNK)], ew_v)
        pltpu.async_copy(h_hbm.at[src_v], rows_v, sem).wait()

        for g in range(GROUPS):
            ew_g = ew_v[pl.ds(g * L, L)]

            @pl.loop(0, L)
            def _edge(t):
                w_b = ew_g.at[jnp.full((L,), t, jnp.int32)].get(
                    mode="promise_in_bounds")
                r = g * L + t
                for j in range(H // L):
                    sl = pl.ds(j * L, L)
                    rows_v[r, sl] = rows_v[r, sl] * w_b

        pltpu.sync_copy(rows_v, acc_sh.at[dst_v], add=True)

    plsc.subcore_barrier()

    for k in range(N_SUB // ZROWS):
        sl = pl.ds(sid * N_SUB + k * ZROWS, ZROWS)
        pltpu.sync_copy(acc_sh.at[sl], stage_v)
        pltpu.sync_copy(stage_v, out_hbm.at[cid].at[sl])


_spmm = functools.partial(
    pl.kernel,
    out_type=jax.ShapeDtypeStruct((NC, N_PAD, H), jnp.float32),
    mesh=plsc.VectorSubcoreMesh(core_axis_name="c", subcore_axis_name="s",
                                num_cores=NC, num_subcores=NS),
    scratch_types=[
        pltpu.VMEM((CHUNK,), jnp.int32),
        pltpu.VMEM((CHUNK,), jnp.int32),
        pltpu.VMEM((CHUNK,), jnp.float32),
        pltpu.VMEM((CHUNK, H), jnp.float32),
        pltpu.VMEM((ZROWS, H), jnp.float32),
        pltpu.VMEM_SHARED((N_PAD, H), jnp.float32),
        pltpu.SemaphoreType.DMA,
    ],
)(_spmm_body)


def _lin_body(x_ref, w_ref, b_ref, o_ref):
    o_ref[...] = jnp.dot(x_ref[...], w_ref[...],
                         preferred_element_type=jnp.float32) + b_ref[...]


_lin = pl.pallas_call(
    _lin_body, out_shape=jax.ShapeDtypeStruct((N, H), jnp.float32))


def _mid_body(p_ref, g_ref, be_ref, m_ref, v_ref, w_ref, b_ref, o_ref):
    agg = p_ref[0, :N] + p_ref[1, :N]
    xb = g_ref[...] * (agg - m_ref[...]) * lax.rsqrt(v_ref[...] + 1e-5) \
        + be_ref[...]
    x1 = jnp.maximum(xb, 0.0)
    o_ref[...] = jnp.dot(x1, w_ref[...],
                         preferred_element_type=jnp.float32) + b_ref[...]


_mid = pl.pallas_call(
    _mid_body, out_shape=jax.ShapeDtypeStruct((N, H), jnp.float32))


def _sum2_body(p_ref, o_ref):
    o_ref[...] = p_ref[0, :N] + p_ref[1, :N]


_sum2 = pl.pallas_call(
    _sum2_body, out_shape=jax.ShapeDtypeStruct((N, H), jnp.float32))


def kernel(x, edge_index, edge_weight, W0, b0, gamma0, beta0, mean0, var0,
           W1, b1):
    src = edge_index[0]
    dst = edge_index[1]
    h = _lin(x, W0, b0.reshape(1, H))
    p = _spmm(h, src, dst, edge_weight)
    h1 = _mid(p, gamma0.reshape(1, H), beta0.reshape(1, H),
              mean0.reshape(1, H), var0.reshape(1, H), W1, b1.reshape(1, H))
    q = _spmm(h1, src, dst, edge_weight)
    return _sum2(q)


# speedup vs baseline: 10.2292x; 2.6103x over previous
import functools

import jax
import jax.numpy as jnp
from jax import lax
from jax.experimental import pallas as pl
from jax.experimental.pallas import tpu as pltpu
from jax.experimental.pallas import tpu_sc as plsc

N = 10000
E = 320000
D = 128
H = 128

NC = 2
NS = 16
L = 16

E_CORE = E // NC
E_SUB = E_CORE // NS
CHUNK = 80
NCHUNK = E_SUB // CHUNK
GROUPS = CHUNK // L
N_PAD = 10240
N_SUB = N_PAD // NS
SLAB = 25
NSLAB = NCHUNK // SLAB


def _spmm_body(h_hbm, src_hbm, dst_hbm, ew_hbm, out_hbm,
               src2_v, dst2_v, ew2_v, rows0, rows1, rows2, acc_sh,
               g0, g1, g2, s0, s1, s2):
    cid = lax.axis_index("c")
    sid = lax.axis_index("s")
    wid = cid * NS + sid

    bufs = (rows0, rows1, rows2)
    gsems = (g0, g1, g2)
    ssems = (s0, s1, s2)

    zvec = jnp.zeros((L,), jnp.float32)

    @pl.loop(0, CHUNK)
    def _zero_stage(r):
        for j in range(H // L):
            rows0[r, pl.ds(j * L, L)] = zvec

    for k in range(N_SUB // CHUNK):
        pltpu.sync_copy(rows0,
                        acc_sh.at[pl.ds(sid * N_SUB + k * CHUNK, CHUNK)])
    plsc.subcore_barrier()

    def issue_gather(j, t):
        pltpu.async_copy(h_hbm.at[src2_v.at[j]], bufs[t], gsems[t])

    def wait_gather(j, t):
        pltpu.make_async_copy(h_hbm.at[src2_v.at[j]], bufs[t],
                              gsems[t]).wait()

    def issue_scatter(j, t):
        pltpu.async_copy(bufs[t], acc_sh.at[dst2_v.at[j]], ssems[t],
                         add=True)

    def wait_scatter(j, t):
        pltpu.make_async_copy(bufs[t], acc_sh.at[dst2_v.at[j]],
                              ssems[t]).wait()

    def scale(j, t):
        rows = bufs[t]
        for g in range(GROUPS):
            ew_g = ew2_v[j, pl.ds(g * L, L)]

            @pl.loop(0, L, unroll=2)
            def _edge(tt):
                w_b = ew_g.at[jnp.full((L,), tt, jnp.int32)].get(
                    mode="promise_in_bounds")
                r = g * L + tt
                for jj in range(H // L):
                    sl = pl.ds(jj * L, L)
                    rows[r, sl] = rows[r, sl] * w_b

    NTRIP = (SLAB - 2 - 2) // 3

    @pl.loop(0, NSLAB)
    def _slab(s):
        pltpu.sync_copy(src_hbm.at[wid].at[s], src2_v)
        pltpu.sync_copy(dst_hbm.at[wid].at[s], dst2_v)
        pltpu.sync_copy(ew_hbm.at[wid].at[s], ew2_v)

        issue_gather(0, 0)
        issue_gather(1, 1)

        @pl.loop(0, NTRIP)
        def _trip(i):
            k3 = 3 * i
            for t in range(3):
                kk = k3 + t
                nk = kk + 2
                nt = (t + 2) % 3

                @pl.when(nk >= 3)
                def _():
                    wait_scatter(nk - 3, nt)

                issue_gather(nk, nt)
                wait_gather(kk, t)
                scale(kk, t)
                issue_scatter(kk, t)

        wait_scatter(20, 2)
        issue_gather(23, 2)
        wait_gather(21, 0)
        scale(21, 0)
        issue_scatter(21, 0)

        wait_scatter(21, 0)
        issue_gather(24, 0)
        wait_gather(22, 1)
        scale(22, 1)
        issue_scatter(22, 1)

        wait_gather(23, 2)
        scale(23, 2)
        issue_scatter(23, 2)

        wait_gather(24, 0)
        scale(24, 0)
        issue_scatter(24, 0)

        wait_scatter(22, 1)
        wait_scatter(23, 2)
        wait_scatter(24, 0)

    plsc.subcore_barrier()

    for k in range(N_SUB // CHUNK):
        sl = pl.ds(sid * N_SUB + k * CHUNK, CHUNK)
        pltpu.sync_copy(acc_sh.at[sl], rows0)
        pltpu.sync_copy(rows0, out_hbm.at[cid].at[sl])


_spmm = functools.partial(
    pl.kernel,
    out_type=jax.ShapeDtypeStruct((NC, N_PAD, H), jnp.float32),
    mesh=plsc.VectorSubcoreMesh(core_axis_name="c", subcore_axis_name="s",
                                num_cores=NC, num_subcores=NS),
    scratch_types=[
        pltpu.VMEM((SLAB, CHUNK), jnp.int32),
        pltpu.VMEM((SLAB, CHUNK), jnp.int32),
        pltpu.VMEM((SLAB, CHUNK), jnp.float32),
        pltpu.VMEM((CHUNK, H), jnp.float32),
        pltpu.VMEM((CHUNK, H), jnp.float32),
        pltpu.VMEM((CHUNK, H), jnp.float32),
        pltpu.VMEM_SHARED((N_PAD, H), jnp.float32),
        pltpu.SemaphoreType.DMA,
        pltpu.SemaphoreType.DMA,
        pltpu.SemaphoreType.DMA,
        pltpu.SemaphoreType.DMA,
        pltpu.SemaphoreType.DMA,
        pltpu.SemaphoreType.DMA,
    ],
)(_spmm_body)


def _lin_body(x_ref, w_ref, b_ref, o_ref):
    o_ref[...] = jnp.dot(x_ref[...], w_ref[...],
                         preferred_element_type=jnp.float32) + b_ref[...]


_lin = pl.pallas_call(
    _lin_body, out_shape=jax.ShapeDtypeStruct((N, H), jnp.float32))


def _mid_body(p_ref, g_ref, be_ref, m_ref, v_ref, w_ref, b_ref, o_ref):
    agg = p_ref[0, :N] + p_ref[1, :N]
    xb = g_ref[...] * (agg - m_ref[...]) * lax.rsqrt(v_ref[...] + 1e-5) \
        + be_ref[...]
    x1 = jnp.maximum(xb, 0.0)
    o_ref[...] = jnp.dot(x1, w_ref[...],
                         preferred_element_type=jnp.float32) + b_ref[...]


_mid = pl.pallas_call(
    _mid_body, out_shape=jax.ShapeDtypeStruct((N, H), jnp.float32))


def _sum2_body(p_ref, o_ref):
    o_ref[...] = p_ref[0, :N] + p_ref[1, :N]


_sum2 = pl.pallas_call(
    _sum2_body, out_shape=jax.ShapeDtypeStruct((N, H), jnp.float32))


def kernel(x, edge_index, edge_weight, W0, b0, gamma0, beta0, mean0, var0,
           W1, b1):
    src = edge_index[0].reshape(NC * NS, NSLAB, SLAB, CHUNK)
    dst = edge_index[1].reshape(NC * NS, NSLAB, SLAB, CHUNK)
    edge_weight = edge_weight.reshape(NC * NS, NSLAB, SLAB, CHUNK)
    h = _lin(x, W0, b0.reshape(1, H))
    p = _spmm(h, src, dst, edge_weight)
    h1 = _mid(p, gamma0.reshape(1, H), beta0.reshape(1, H),
              mean0.reshape(1, H), var0.reshape(1, H), W1, b1.reshape(1, H))
    q = _spmm(h1, src, dst, edge_weight)
    return _sum2(q)


# speedup vs baseline: 10.4212x; 1.0188x over previous
import functools

import jax
import jax.numpy as jnp
from jax import lax
from jax.experimental import pallas as pl
from jax.experimental.pallas import tpu as pltpu
from jax.experimental.pallas import tpu_sc as plsc

N = 10000
E = 320000
D = 128
H = 128

NC = 2
NS = 16
L = 16

E_CORE = E // NC
E_SUB = E_CORE // NS
CHUNK = 80
NCHUNK = E_SUB // CHUNK
GROUPS = CHUNK // L
N_PAD = 10240
N_SUB = N_PAD // NS
SLAB = 25
NSLAB = NCHUNK // SLAB


def _spmm_body(h_hbm, src_hbm, dst_hbm, ew_hbm, out_hbm,
               src2_v, dst2_v, ew2_v, rows0, rows1, rows2, acc_sh,
               g0, g1, g2, s0, s1, s2):
    cid = lax.axis_index("c")
    sid = lax.axis_index("s")
    wid = cid * NS + sid

    bufs = (rows0, rows1, rows2)
    gsems = (g0, g1, g2)
    ssems = (s0, s1, s2)

    zvec = jnp.zeros((L,), jnp.float32)

    @pl.loop(0, CHUNK)
    def _zero_stage(r):
        for j in range(H // L):
            rows0[r, pl.ds(j * L, L)] = zvec

    for k in range(N_SUB // CHUNK):
        pltpu.async_copy(rows0,
                         acc_sh.at[pl.ds(sid * N_SUB + k * CHUNK, CHUNK)],
                         g0)
    for k in range(N_SUB // CHUNK):
        pltpu.make_async_copy(
            rows0, acc_sh.at[pl.ds(sid * N_SUB + k * CHUNK, CHUNK)],
            g0).wait()
    plsc.subcore_barrier()

    def issue_gather(j, t):
        pltpu.async_copy(h_hbm.at[src2_v.at[j]], bufs[t], gsems[t])

    def wait_gather(j, t):
        pltpu.make_async_copy(h_hbm.at[src2_v.at[j]], bufs[t],
                              gsems[t]).wait()

    def issue_scatter(j, t):
        pltpu.async_copy(bufs[t], acc_sh.at[dst2_v.at[j]], ssems[t],
                         add=True)

    def wait_scatter(j, t):
        pltpu.make_async_copy(bufs[t], acc_sh.at[dst2_v.at[j]],
                              ssems[t]).wait()

    def scale(j, t):
        rows = bufs[t]

        @pl.loop(0, GROUPS)
        def _grp(g):
            gL = g * L
            ew_g = ew2_v[j, pl.ds(gL, L)]
            for tt in range(L):
                w_b = ew_g.at[jnp.full((L,), tt, jnp.int32)].get(
                    mode="promise_in_bounds")
                r = gL + tt
                for jj in range(H // L):
                    sl = pl.ds(jj * L, L)
                    rows[r, sl] = rows[r, sl] * w_b

    NTRIP = (SLAB - 2 - 2) // 3

    @pl.loop(0, NSLAB)
    def _slab(s):
        pltpu.sync_copy(src_hbm.at[wid].at[s], src2_v)
        pltpu.sync_copy(dst_hbm.at[wid].at[s], dst2_v)
        pltpu.sync_copy(ew_hbm.at[wid].at[s], ew2_v)

        issue_gather(0, 0)
        issue_gather(1, 1)

        @pl.loop(0, NTRIP)
        def _trip(i):
            k3 = 3 * i
            for t in range(3):
                kk = k3 + t
                nk = kk + 2
                nt = (t + 2) % 3

                @pl.when(nk >= 3)
                def _():
                    wait_scatter(nk - 3, nt)

                issue_gather(nk, nt)
                wait_gather(kk, t)
                scale(kk, t)
                issue_scatter(kk, t)

        wait_scatter(20, 2)
        issue_gather(23, 2)
        wait_gather(21, 0)
        scale(21, 0)
        issue_scatter(21, 0)

        wait_scatter(21, 0)
        issue_gather(24, 0)
        wait_gather(22, 1)
        scale(22, 1)
        issue_scatter(22, 1)

        wait_gather(23, 2)
        scale(23, 2)
        issue_scatter(23, 2)

        wait_gather(24, 0)
        scale(24, 0)
        issue_scatter(24, 0)

        wait_scatter(22, 1)
        wait_scatter(23, 2)
        wait_scatter(24, 0)

    plsc.subcore_barrier()

    nwr = N_SUB // CHUNK
    sls = [pl.ds(sid * N_SUB + k * CHUNK, CHUNK) for k in range(nwr)]
    for k in range(nwr):
        b = k % 2
        if k >= 2:
            pltpu.make_async_copy(bufs[b], out_hbm.at[cid].at[sls[k - 2]],
                                  gsems[b]).wait()
        pltpu.sync_copy(acc_sh.at[sls[k]], bufs[b])
        pltpu.async_copy(bufs[b], out_hbm.at[cid].at[sls[k]], gsems[b])
    for k in range(nwr - 2, nwr):
        pltpu.make_async_copy(bufs[k % 2], out_hbm.at[cid].at[sls[k]],
                              gsems[k % 2]).wait()


_spmm = functools.partial(
    pl.kernel,
    out_type=jax.ShapeDtypeStruct((NC, N_PAD, H), jnp.float32),
    mesh=plsc.VectorSubcoreMesh(core_axis_name="c", subcore_axis_name="s",
                                num_cores=NC, num_subcores=NS),
    scratch_types=[
        pltpu.VMEM((SLAB, CHUNK), jnp.int32),
        pltpu.VMEM((SLAB, CHUNK), jnp.int32),
        pltpu.VMEM((SLAB, CHUNK), jnp.float32),
        pltpu.VMEM((CHUNK, H), jnp.float32),
        pltpu.VMEM((CHUNK, H), jnp.float32),
        pltpu.VMEM((CHUNK, H), jnp.float32),
        pltpu.VMEM_SHARED((N_PAD, H), jnp.float32),
        pltpu.SemaphoreType.DMA,
        pltpu.SemaphoreType.DMA,
        pltpu.SemaphoreType.DMA,
        pltpu.SemaphoreType.DMA,
        pltpu.SemaphoreType.DMA,
        pltpu.SemaphoreType.DMA,
    ],
)(_spmm_body)


def _lin_body(x_ref, w_ref, b_ref, o_ref):
    o_ref[...] = jnp.dot(x_ref[...], w_ref[...],
                         preferred_element_type=jnp.float32) + b_ref[...]


_lin = pl.pallas_call(
    _lin_body, out_shape=jax.ShapeDtypeStruct((N, H), jnp.float32))


def _mid_body(p_ref, g_ref, be_ref, m_ref, v_ref, w_ref, b_ref, o_ref):
    agg = p_ref[0, :N] + p_ref[1, :N]
    xb = g_ref[...] * (agg - m_ref[...]) * lax.rsqrt(v_ref[...] + 1e-5) \
        + be_ref[...]
    x1 = jnp.maximum(xb, 0.0)
    o_ref[...] = jnp.dot(x1, w_ref[...],
                         preferred_element_type=jnp.float32) + b_ref[...]


_mid = pl.pallas_call(
    _mid_body, out_shape=jax.ShapeDtypeStruct((N, H), jnp.float32))


def _sum2_body(p_ref, o_ref):
    o_ref[...] = p_ref[0, :N] + p_ref[1, :N]


_sum2 = pl.pallas_call(
    _sum2_body, out_shape=jax.ShapeDtypeStruct((N, H), jnp.float32))


def kernel(x, edge_index, edge_weight, W0, b0, gamma0, beta0, mean0, var0,
           W1, b1):
    src = edge_index[0].reshape(NC * NS, NSLAB, SLAB, CHUNK)
    dst = edge_index[1].reshape(NC * NS, NSLAB, SLAB, CHUNK)
    edge_weight = edge_weight.reshape(NC * NS, NSLAB, SLAB, CHUNK)
    h = _lin(x, W0, b0.reshape(1, H))
    p = _spmm(h, src, dst, edge_weight)
    h1 = _mid(p, gamma0.reshape(1, H), beta0.reshape(1, H),
              mean0.reshape(1, H), var0.reshape(1, H), W1, b1.reshape(1, H))
    q = _spmm(h1, src, dst, edge_weight)
    return _sum2(q)
